# Initial kernel scaffold; baseline (speedup 1.0000x reference)
import math

import jax
import jax.numpy as jnp
from jax import lax
from jax.experimental import pallas as pl
from jax.experimental.pallas import tpu as pltpu
from jax.experimental.pallas import tpu_sc as plsc

VOCAB = 1000000
NCAT = 16
B = 4096
L = 200
K = 50

NC = 2
NS = 16
NW = NC * NS
UPW = B // NW

IW = 128
HIST_ROWS_PER_W = B * L // IW // NW
REC_ROWS_PER_W = B * K // IW // NW
CHUNK_ROWS = 25
HIST_CHUNKS = HIST_ROWS_PER_W // CHUNK_ROWS
REC_CHUNKS = REC_ROWS_PER_W // CHUNK_ROWS
CHUNK_IDX = CHUNK_ROWS * IW


def _sc_body(table, us2, rec2, p_out, q_out, idx_v, rows_v, psum_v, qsum_v, sem):
    c = lax.axis_index("c")
    s = lax.axis_index("s")
    wid = c * NS + s

    zero = jnp.zeros((16,), jnp.int32)

    def gather_chunk(src_hbm, row_base):
        pltpu.sync_copy(src_hbm.at[pl.ds(row_base, CHUNK_ROWS)], idx_v)
        copies = [
            pltpu.make_async_copy(
                table.at[idx_v.at[j]],
                rows_v.at[pl.ds(j * IW, IW)],
                sem,
            )
            for j in range(CHUNK_ROWS)
        ]
        for cp in copies:
            cp.start()
        for cp in copies:
            cp.wait()

    def accum(run_len, n_users, out_ref, out_base):
        def u_body(u, _):
            base = u * run_len

            def r_body(r, accs):
                a0, a1 = accs
                t = base + r * 2
                return (a0 + rows_v[t], a1 + rows_v[t + 1])

            a0, a1 = lax.fori_loop(0, run_len // 2, r_body, (zero, zero))
            out_ref[out_base + u] = a0 + a1
            return 0

        lax.fori_loop(0, n_users, u_body, 0)

    def hist_chunk(ch, _):
        gather_chunk(us2, wid * HIST_ROWS_PER_W + ch * CHUNK_ROWS)
        accum(L, CHUNK_IDX // L, psum_v, ch * (CHUNK_IDX // L))
        return 0

    lax.fori_loop(0, HIST_CHUNKS, hist_chunk, 0)

    def rec_chunk(ch, _):
        gather_chunk(rec2, wid * REC_ROWS_PER_W + ch * CHUNK_ROWS)
        accum(K, CHUNK_IDX // K, qsum_v, ch * (CHUNK_IDX // K))
        return 0

    lax.fori_loop(0, REC_CHUNKS, rec_chunk, 0)

    pltpu.sync_copy(psum_v, p_out.at[pl.ds(wid * UPW, UPW)])
    pltpu.sync_copy(qsum_v, q_out.at[pl.ds(wid * UPW, UPW)])


_sc_sums = pl.kernel(
    _sc_body,
    out_type=(
        jax.ShapeDtypeStruct((B, NCAT), jnp.int32),
        jax.ShapeDtypeStruct((B, NCAT), jnp.int32),
    ),
    mesh=plsc.VectorSubcoreMesh(
        core_axis_name="c", subcore_axis_name="s", num_cores=NC, num_subcores=NS
    ),
    scratch_types=[
        pltpu.VMEM((CHUNK_ROWS, IW), jnp.int32),
        pltpu.VMEM((CHUNK_IDX, NCAT), jnp.int32),
        pltpu.VMEM((UPW, NCAT), jnp.int32),
        pltpu.VMEM((UPW, NCAT), jnp.int32),
        pltpu.SemaphoreType.DMA,
    ],
)


def _hell_body(p_ref, q_ref, o_ref):
    p = p_ref[...].astype(jnp.float32) * (1.0 / L)
    q = q_ref[...].astype(jnp.float32) * (1.0 / K)
    d = jnp.sqrt(p) - jnp.sqrt(q)
    o_ref[...] = jnp.sum(d * d, axis=1, keepdims=True) * (1.0 / math.sqrt(2.0))


_hell = pl.pallas_call(
    _hell_body,
    grid=(8,),
    in_specs=[
        pl.BlockSpec((B // 8, NCAT), lambda i: (i, 0)),
        pl.BlockSpec((B // 8, NCAT), lambda i: (i, 0)),
    ],
    out_specs=pl.BlockSpec((B // 8, 1), lambda i: (i, 0)),
    out_shape=jax.ShapeDtypeStruct((B, 1), jnp.float32),
)


@jax.jit
def _impl(item_categories, user_sequence, recommendations):
    us2 = user_sequence.reshape(B * L // IW, IW)
    rec2 = recommendations.reshape(B * K // IW, IW)
    p_sum, q_sum = _sc_sums(item_categories, us2, rec2)
    return _hell(p_sum, q_sum).reshape(B)


def kernel(item_categories, user_sequence, recommendations):
    return _impl(item_categories, user_sequence, recommendations)


# speedup vs baseline: 1.4799x; 1.4799x over previous
import math

import jax
import jax.numpy as jnp
from jax import lax
from jax.experimental import pallas as pl
from jax.experimental.pallas import tpu as pltpu
from jax.experimental.pallas import tpu_sc as plsc

VOCAB = 1000000
NCAT = 16
B = 4096
L = 200
K = 50

NC = 2
NS = 16
NW = NC * NS
UPW = B // NW

IW = 128
HIST_ROWS_PER_W = B * L // IW // NW
REC_ROWS_PER_W = B * K // IW // NW
CHUNK_ROWS = 25
HIST_CHUNKS = HIST_ROWS_PER_W // CHUNK_ROWS
REC_CHUNKS = REC_ROWS_PER_W // CHUNK_ROWS
CHUNK_IDX = CHUNK_ROWS * IW


def _sc_body(table, us3, rec3, p_out, q_out, idx_h, idx_r, rows_v, psum_v, qsum_v, sem):
    c = lax.axis_index("c")
    s = lax.axis_index("s")
    wid = c * NS + s

    zero = jnp.zeros((16,), jnp.int32)

    pltpu.sync_copy(us3.at[wid], idx_h)
    pltpu.sync_copy(rec3.at[wid], idx_r)

    def gather_chunk(idx_ref, row_base):
        copies = [
            pltpu.make_async_copy(
                table.at[idx_ref.at[row_base + j]],
                rows_v.at[pl.ds(j * IW, IW)],
                sem,
            )
            for j in range(CHUNK_ROWS)
        ]
        for cp in copies:
            cp.start()
        for cp in copies:
            cp.wait()

    def accum(run_len, n_users, out_ref, out_base):
        def u_body(u, _):
            base = u * run_len

            def r_body(r, accs):
                a0, a1 = accs
                t = base + r * 2
                return (a0 + rows_v[t], a1 + rows_v[t + 1])

            a0, a1 = lax.fori_loop(0, run_len // 2, r_body, (zero, zero))
            out_ref[out_base + u] = a0 + a1
            return 0

        lax.fori_loop(0, n_users, u_body, 0)

    def hist_chunk(ch, _):
        gather_chunk(idx_h, ch * CHUNK_ROWS)
        accum(L, CHUNK_IDX // L, psum_v, ch * (CHUNK_IDX // L))
        return 0

    lax.fori_loop(0, HIST_CHUNKS, hist_chunk, 0)

    def rec_chunk(ch, _):
        gather_chunk(idx_r, ch * CHUNK_ROWS)
        accum(K, CHUNK_IDX // K, qsum_v, ch * (CHUNK_IDX // K))
        return 0

    lax.fori_loop(0, REC_CHUNKS, rec_chunk, 0)

    pltpu.sync_copy(psum_v, p_out.at[pl.ds(wid * UPW, UPW)])
    pltpu.sync_copy(qsum_v, q_out.at[pl.ds(wid * UPW, UPW)])


_sc_sums = pl.kernel(
    _sc_body,
    out_type=(
        jax.ShapeDtypeStruct((B, NCAT), jnp.int32),
        jax.ShapeDtypeStruct((B, NCAT), jnp.int32),
    ),
    mesh=plsc.VectorSubcoreMesh(
        core_axis_name="c", subcore_axis_name="s", num_cores=NC, num_subcores=NS
    ),
    scratch_types=[
        pltpu.VMEM((HIST_ROWS_PER_W, IW), jnp.int32),
        pltpu.VMEM((REC_ROWS_PER_W, IW), jnp.int32),
        pltpu.VMEM((CHUNK_IDX, NCAT), jnp.int32),
        pltpu.VMEM((UPW, NCAT), jnp.int32),
        pltpu.VMEM((UPW, NCAT), jnp.int32),
        pltpu.SemaphoreType.DMA,
    ],
    compiler_params=pltpu.CompilerParams(use_tc_tiling_on_sc=False),
)


def _hell_body(p_ref, q_ref, o_ref):
    p = p_ref[...].astype(jnp.float32) * (1.0 / L)
    q = q_ref[...].astype(jnp.float32) * (1.0 / K)
    d = jnp.sqrt(p) - jnp.sqrt(q)
    o_ref[...] = jnp.sum(d * d, axis=1, keepdims=True) * (1.0 / math.sqrt(2.0))


_hell = pl.pallas_call(
    _hell_body,
    grid=(8,),
    in_specs=[
        pl.BlockSpec((B // 8, NCAT), lambda i: (i, 0)),
        pl.BlockSpec((B // 8, NCAT), lambda i: (i, 0)),
    ],
    out_specs=pl.BlockSpec((B // 8, 1), lambda i: (i, 0)),
    out_shape=jax.ShapeDtypeStruct((B, 1), jnp.float32),
)


@jax.jit
def _impl(item_categories, user_sequence, recommendations):
    us3 = user_sequence.reshape(NW, HIST_ROWS_PER_W, IW)
    rec3 = recommendations.reshape(NW, REC_ROWS_PER_W, IW)
    p_sum, q_sum = _sc_sums(item_categories, us3, rec3)
    return _hell(p_sum, q_sum).reshape(B)


def kernel(item_categories, user_sequence, recommendations):
    return _impl(item_categories, user_sequence, recommendations)
